# baseline (speedup 1.0000x reference)
import functools

import jax
import jax.numpy as jnp
from jax.experimental import pallas as pl
from jax.experimental.pallas import tpu as pltpu

_NB = 32


def _pool2x2(y):
    nb, h, w, c = y.shape
    y6 = y.reshape(nb, h // 2, 2, w // 2, 2, c)
    yw = jnp.maximum(y6[:, :, :, :, 0, :], y6[:, :, :, :, 1, :])
    return jnp.maximum(yw[:, :, 0, :, :], yw[:, :, 1, :, :])


def _fused_body(x_ref, w1b_ref, b1_ref, w2_ref, b2_ref, w3_ref, b3_ref,
                wl1_ref, bl1_ref, wl2_ref, bl2_ref, o_ref, *, nb):
    f32 = jnp.float32
    bf16 = jnp.bfloat16
    x = x_ref[...]
    xp = jnp.pad(x, ((0, 0), (1, 1), (1, 1)))

    xc = jnp.concatenate(
        [xp[:, 0:28, :].reshape(nb * 28, 30),
         xp[:, 1:29, :].reshape(nb * 28, 30),
         xp[:, 2:30, :].reshape(nb * 28, 30)], axis=1)
    y1 = jnp.dot(xc, w1b_ref[...], preferred_element_type=f32)
    y1 = y1.reshape(nb, 28, 28, 32)
    h1 = jnp.maximum(_pool2x2(y1) + b1_ref[0], 0.0).astype(bf16)

    hp = jnp.pad(h1, ((0, 0), (1, 1), (1, 1), (0, 0)))
    p2 = jnp.concatenate(
        [hp[:, dy:dy + 14, dx:dx + 14, :].reshape(nb * 196, 32)
         for dy in range(3) for dx in range(3)], axis=1)
    y2 = jnp.dot(p2, w2_ref[...], preferred_element_type=f32)
    y2 = y2.reshape(nb, 14, 14, 64)
    h2 = jnp.maximum(_pool2x2(y2) + b2_ref[0], 0.0).astype(bf16)

    hp3 = jnp.pad(h2, ((0, 0), (1, 1), (1, 1), (0, 0)))
    p3 = jnp.concatenate(
        [hp3[:, dy:dy + 7, dx:dx + 7, :].reshape(nb * 49, 64)
         for dy in range(3) for dx in range(3)], axis=1)
    y3 = jnp.dot(p3, w3_ref[...], preferred_element_type=f32)
    y3 = y3.reshape(nb, 7, 7, 64)[:, 0:6, 0:6, :]
    h3 = jnp.maximum(_pool2x2(y3) + b3_ref[0], 0.0).astype(bf16)

    f = h3.reshape(nb, 576)
    z = jnp.dot(f, wl1_ref[...], preferred_element_type=f32)
    z = jnp.maximum(z + bl1_ref[0], 0.0).astype(bf16)
    o_ref[...] = jnp.dot(z, wl2_ref[...],
                         preferred_element_type=f32) + bl2_ref[0]


def _band_w1(w1):
    w = w1.reshape(3, 3, 32)
    b = jnp.zeros((3, 30, 28, 32), w1.dtype)
    iw = jnp.arange(28)
    for dy in range(3):
        for dx in range(3):
            b = b.at[dy, iw + dx, iw, :].set(w[dy, dx][None, :])
    return b.reshape(90, 28 * 32)


def kernel(x_nchw, w1, b1, w2, b2, w3, b3, wl1, bl1, wl2, bl2):
    n = x_nchw.shape[0]
    bf16 = jnp.bfloat16
    x = x_nchw.reshape(n, 28, 28).astype(bf16)

    w1b = _band_w1(w1).astype(bf16)
    w2m = w2.reshape(288, 64).astype(bf16)
    w3m = w3.reshape(576, 64).astype(bf16)
    wl1f = wl1.reshape(64, 3, 3, 128).transpose(1, 2, 0, 3)
    wl1f = wl1f.reshape(576, 128).astype(bf16)
    wl2p = jnp.pad(wl2, ((0, 0), (0, 118))).astype(bf16)
    b1r = b1.reshape(1, 32)
    b2r = b2.reshape(1, 64)
    b3r = b3.reshape(1, 64)
    bl1r = bl1.reshape(1, 128)
    bl2r = jnp.pad(bl2, (0, 118)).reshape(1, 128)

    nb = _NB
    grid = n // nb
    macs = n * (28 * 90 * 896 + 196 * 288 * 64 + 49 * 576 * 64
                + 576 * 128 + 128 * 128)
    cost = pl.CostEstimate(
        flops=2 * macs, transcendentals=0,
        bytes_accessed=4 * (n * 28 * 28 + n * 128 + 90 * 896 + 288 * 64
                            + 576 * 64 + 576 * 128 + 128 * 128))

    out = pl.pallas_call(
        functools.partial(_fused_body, nb=nb),
        out_shape=jax.ShapeDtypeStruct((n, 128), jnp.float32),
        grid=(grid,),
        in_specs=[
            pl.BlockSpec((nb, 28, 28), lambda i: (i, 0, 0)),
            pl.BlockSpec((90, 896), lambda i: (0, 0)),
            pl.BlockSpec((1, 32), lambda i: (0, 0)),
            pl.BlockSpec((288, 64), lambda i: (0, 0)),
            pl.BlockSpec((1, 64), lambda i: (0, 0)),
            pl.BlockSpec((576, 64), lambda i: (0, 0)),
            pl.BlockSpec((1, 64), lambda i: (0, 0)),
            pl.BlockSpec((576, 128), lambda i: (0, 0)),
            pl.BlockSpec((1, 128), lambda i: (0, 0)),
            pl.BlockSpec((128, 128), lambda i: (0, 0)),
            pl.BlockSpec((1, 128), lambda i: (0, 0)),
        ],
        out_specs=pl.BlockSpec((nb, 128), lambda i: (i, 0)),
        compiler_params=pltpu.CompilerParams(
            dimension_semantics=("parallel",),
            vmem_limit_bytes=64 * 1024 * 1024),
        cost_estimate=cost,
    )(x, w1b, b1r, w2m, b2r, w3m, b3r, wl1f, bl1r, wl2p, bl2r)
    return out[:, :10]


# speedup vs baseline: 1.1917x; 1.1917x over previous
import functools

import jax
import jax.numpy as jnp
from jax.experimental import pallas as pl
from jax.experimental.pallas import tpu as pltpu

_NB = 32


def _pool2x2(y):
    nb, h, w, c = y.shape
    y6 = y.reshape(nb, h // 2, 2, w // 2, 2, c)
    yw = jnp.maximum(y6[:, :, :, :, 0, :], y6[:, :, :, :, 1, :])
    return jnp.maximum(yw[:, :, 0, :, :], yw[:, :, 1, :, :])


def _fused_body(x_ref, w1b_ref, b1_ref, w2_ref, b2_ref, w3_ref, b3_ref,
                wl1_ref, bl1_ref, wl2_ref, bl2_ref, o_ref, *, nb):
    f32 = jnp.float32
    x = x_ref[...]
    xp = jnp.pad(x, ((0, 0), (1, 1), (1, 1)))

    xc = jnp.concatenate(
        [xp[:, 0:28, :].reshape(nb * 28, 30),
         xp[:, 1:29, :].reshape(nb * 28, 30),
         xp[:, 2:30, :].reshape(nb * 28, 30)], axis=1)
    y1 = jnp.dot(xc, w1b_ref[...], preferred_element_type=f32)
    y1 = y1.reshape(nb, 28, 28, 32)
    h1 = jnp.maximum(_pool2x2(y1) + b1_ref[0], 0.0)

    hp = jnp.pad(h1, ((0, 0), (1, 1), (1, 1), (0, 0)))
    p2 = jnp.concatenate(
        [hp[:, dy:dy + 14, dx:dx + 14, :].reshape(nb * 196, 32)
         for dy in range(3) for dx in range(3)], axis=1)
    y2 = jnp.dot(p2, w2_ref[...], preferred_element_type=f32)
    y2 = y2.reshape(nb, 14, 14, 64)
    h2 = jnp.maximum(_pool2x2(y2) + b2_ref[0], 0.0)

    hp3 = jnp.pad(h2, ((0, 0), (1, 1), (1, 1), (0, 0)))
    p3 = jnp.concatenate(
        [hp3[:, dy:dy + 7, dx:dx + 7, :].reshape(nb * 49, 64)
         for dy in range(3) for dx in range(3)], axis=1)
    y3 = jnp.dot(p3, w3_ref[...], preferred_element_type=f32)
    y3 = y3.reshape(nb, 7, 7, 64)[:, 0:6, 0:6, :]
    h3 = jnp.maximum(_pool2x2(y3) + b3_ref[0], 0.0)

    f = h3.reshape(nb, 576)
    z = jnp.dot(f, wl1_ref[...], preferred_element_type=f32)
    z = jnp.maximum(z + bl1_ref[0], 0.0)
    o_ref[...] = jnp.dot(z, wl2_ref[...],
                         preferred_element_type=f32) + bl2_ref[0]


def _band_w1(w1):
    w = w1.reshape(3, 3, 32)
    b = jnp.zeros((3, 30, 28, 32), w1.dtype)
    iw = jnp.arange(28)
    for dy in range(3):
        for dx in range(3):
            b = b.at[dy, iw + dx, iw, :].set(w[dy, dx][None, :])
    return b.reshape(90, 28 * 32)


def kernel(x_nchw, w1, b1, w2, b2, w3, b3, wl1, bl1, wl2, bl2):
    n = x_nchw.shape[0]
    x = x_nchw.reshape(n, 28, 28)

    w1b = _band_w1(w1)
    w2m = w2.reshape(288, 64)
    w3m = w3.reshape(576, 64)
    wl1f = wl1.reshape(64, 3, 3, 128).transpose(1, 2, 0, 3).reshape(576, 128)
    wl2p = jnp.pad(wl2, ((0, 0), (0, 118)))
    b1r = b1.reshape(1, 32)
    b2r = b2.reshape(1, 64)
    b3r = b3.reshape(1, 64)
    bl1r = bl1.reshape(1, 128)
    bl2r = jnp.pad(bl2, (0, 118)).reshape(1, 128)

    nb = _NB
    grid = n // nb
    macs = n * (28 * 90 * 896 + 196 * 288 * 64 + 49 * 576 * 64
                + 576 * 128 + 128 * 128)
    cost = pl.CostEstimate(
        flops=2 * macs, transcendentals=0,
        bytes_accessed=4 * (n * 28 * 28 + n * 128 + 90 * 896 + 288 * 64
                            + 576 * 64 + 576 * 128 + 128 * 128))

    out = pl.pallas_call(
        functools.partial(_fused_body, nb=nb),
        out_shape=jax.ShapeDtypeStruct((n, 128), jnp.float32),
        grid=(grid,),
        in_specs=[
            pl.BlockSpec((nb, 28, 28), lambda i: (i, 0, 0)),
            pl.BlockSpec((90, 896), lambda i: (0, 0)),
            pl.BlockSpec((1, 32), lambda i: (0, 0)),
            pl.BlockSpec((288, 64), lambda i: (0, 0)),
            pl.BlockSpec((1, 64), lambda i: (0, 0)),
            pl.BlockSpec((576, 64), lambda i: (0, 0)),
            pl.BlockSpec((1, 64), lambda i: (0, 0)),
            pl.BlockSpec((576, 128), lambda i: (0, 0)),
            pl.BlockSpec((1, 128), lambda i: (0, 0)),
            pl.BlockSpec((128, 128), lambda i: (0, 0)),
            pl.BlockSpec((1, 128), lambda i: (0, 0)),
        ],
        out_specs=pl.BlockSpec((nb, 128), lambda i: (i, 0)),
        compiler_params=pltpu.CompilerParams(
            dimension_semantics=("parallel",),
            vmem_limit_bytes=64 * 1024 * 1024),
        cost_estimate=cost,
    )(x, w1b, b1r, w2m, b2r, w3m, b3r, wl1f, bl1r, wl2p, bl2r)
    return out[:, :10]
